# Initial kernel scaffold; baseline (speedup 1.0000x reference)
import jax
import jax.numpy as jnp
from jax.experimental import pallas as pl


def kernel(at_no, pos, edge_index, edge_index_full, embed_table, conv_Wrbf, conv_Wself, conv_Wmsg, trans_Wii, trans_Wrbf, trans_Wij, out_Wii, out_Wij):
    raise NotImplementedError("write your pallas kernel here")


# speedup vs baseline: 1.9077x; 1.9077x over previous
import functools

import jax
import jax.numpy as jnp
from jax import lax
from jax.experimental import pallas as pl
from jax.experimental.pallas import tpu as pltpu
from jax.experimental.pallas import tpu_sc as plsc

N = 10000
E = 320000
D = 128
NB = 32
OUT = 64
CUTOFF = 5.0

NC = 2
NS = 16
NW = NC * NS
CH = 80
NACC = 10240

_MESH = dict(core_axis_name="c", subcore_axis_name="s")


def _make_sc_geometry():
    per_w = E // NW
    iters = per_w // 16

    sd = jax.ShapeDtypeStruct((E,), jnp.float32)

    @functools.partial(
        pl.kernel,
        out_type=(sd, sd, sd),
        mesh=plsc.VectorSubcoreMesh(**_MESH),
        compiler_params=pltpu.CompilerParams(needs_layout_passes=False),
        scratch_types=[
            pltpu.VMEM((N,), jnp.float32),
            pltpu.VMEM((N,), jnp.float32),
            pltpu.VMEM((N,), jnp.float32),
            pltpu.VMEM((per_w,), jnp.int32),
            pltpu.VMEM((per_w,), jnp.int32),
            pltpu.VMEM((per_w,), jnp.float32),
            pltpu.VMEM((per_w,), jnp.float32),
        ],
    )
    def k(px_h, py_h, pz_h, src_h, dst_h, srcf_h, dstf_h,
          d2_h, vs_h, d2f_h, px, py, pz, si, di, d2v, vsv):
        wid = lax.axis_index("s") * NC + lax.axis_index("c")
        base = pl.multiple_of(wid * per_w, 8)
        pltpu.sync_copy(px_h, px)
        pltpu.sync_copy(py_h, py)
        pltpu.sync_copy(pz_h, pz)

        def run(write_vsum):
            def body(i, carry):
                off = pl.multiple_of(i * 16, 8)
                s16 = si[pl.ds(off, 16)]
                d16 = di[pl.ds(off, 16)]
                vx = plsc.load_gather(px, [d16]) - plsc.load_gather(px, [s16])
                vy = plsc.load_gather(py, [d16]) - plsc.load_gather(py, [s16])
                vz = plsc.load_gather(pz, [d16]) - plsc.load_gather(pz, [s16])
                d2v[pl.ds(off, 16)] = vx * vx + vy * vy + vz * vz
                if write_vsum:
                    vsv[pl.ds(off, 16)] = vx + vy + vz
                return carry
            lax.fori_loop(0, iters, body, 0)

        pltpu.sync_copy(src_h.at[pl.ds(base, per_w)], si)
        pltpu.sync_copy(dst_h.at[pl.ds(base, per_w)], di)
        run(True)
        pltpu.sync_copy(d2v, d2_h.at[pl.ds(base, per_w)])
        pltpu.sync_copy(vsv, vs_h.at[pl.ds(base, per_w)])

        pltpu.sync_copy(srcf_h.at[pl.ds(base, per_w)], si)
        pltpu.sync_copy(dstf_h.at[pl.ds(base, per_w)], di)
        run(False)
        pltpu.sync_copy(d2v, d2f_h.at[pl.ds(base, per_w)])

    return k


def _make_sc_gather(M):
    per_w = M // NW
    n_chunks = per_w // CH
    assert per_w * NW == M and n_chunks * CH == per_w

    @functools.partial(
        pl.kernel,
        out_type=jax.ShapeDtypeStruct((M, D), jnp.float32),
        mesh=plsc.VectorSubcoreMesh(**_MESH),
        scratch_types=[
            pltpu.VMEM((n_chunks, CH), jnp.int32),
            pltpu.VMEM((2, CH, D), jnp.float32),
            pltpu.SemaphoreType.DMA,
            pltpu.SemaphoreType.DMA,
        ],
    )
    def k(table_hbm, idx_hbm, out_hbm, idx_v, buf_v, gsem, osem):
        wid = lax.axis_index("s") * NC + lax.axis_index("c")
        base = pl.multiple_of(wid * per_w, 8)
        pltpu.sync_copy(idx_hbm.at[wid], idx_v)

        pltpu.async_copy(table_hbm.at[idx_v.at[0]], buf_v.at[0], gsem)

        def body(ck, carry):
            slot = lax.rem(ck, 2)
            nxt = lax.rem(ck + 1, 2)

            @pl.when(jnp.logical_and(ck >= 1, ck + 1 < n_chunks))
            def _():
                pltpu.make_async_copy(
                    buf_v.at[nxt],
                    out_hbm.at[pl.ds(base, CH)], osem).wait()

            @pl.when(ck + 1 < n_chunks)
            def _():
                pltpu.async_copy(
                    table_hbm.at[idx_v.at[ck + 1]], buf_v.at[nxt], gsem)

            pltpu.make_async_copy(
                table_hbm.at[idx_v.at[ck]], buf_v.at[slot], gsem).wait()
            pltpu.async_copy(
                buf_v.at[slot],
                out_hbm.at[pl.ds(base + ck * CH, CH)], osem)
            return carry

        lax.fori_loop(0, n_chunks, body, 0)
        pltpu.make_async_copy(buf_v.at[0], out_hbm.at[pl.ds(base, CH)],
                              osem).wait()
        pltpu.make_async_copy(buf_v.at[1], out_hbm.at[pl.ds(base, CH)],
                              osem).wait()

    return k


def _make_sc_scatter_add():
    per_w = E // NW
    n_chunks = per_w // CH
    rows_per_tile = NACC // NS

    @functools.partial(
        pl.kernel,
        out_type=jax.ShapeDtypeStruct((NC, NACC, D), jnp.float32),
        mesh=plsc.VectorSubcoreMesh(**_MESH),
        scratch_types=[
            pltpu.VMEM_SHARED((NACC, D), jnp.float32),
            pltpu.VMEM((2, CH, D), jnp.float32),
            pltpu.VMEM((n_chunks, CH), jnp.int32),
            pltpu.SemaphoreType.DMA,
        ],
    )
    def k(vals_hbm, idx_hbm, zeros_hbm, out_hbm, acc_sh, vals_v, idx_v, vsem):
        sid = lax.axis_index("s")
        cid = lax.axis_index("c")
        wid = sid * NC + cid
        pltpu.sync_copy(idx_hbm.at[wid], idx_v)
        tbase = pl.multiple_of(sid * rows_per_tile, 8)
        pltpu.sync_copy(zeros_hbm.at[pl.ds(tbase, rows_per_tile)],
                        acc_sh.at[pl.ds(tbase, rows_per_tile)])
        plsc.subcore_barrier()

        base = pl.multiple_of(wid * per_w, 8)
        pltpu.async_copy(vals_hbm.at[pl.ds(base, CH)], vals_v.at[0], vsem)

        def body(ck, carry):
            slot = lax.rem(ck, 2)
            nxt = lax.rem(ck + 1, 2)

            @pl.when(ck + 1 < n_chunks)
            def _():
                pltpu.async_copy(
                    vals_hbm.at[pl.ds(base + (ck + 1) * CH, CH)],
                    vals_v.at[nxt], vsem)

            pltpu.make_async_copy(
                vals_hbm.at[pl.ds(base, CH)], vals_v.at[slot], vsem).wait()
            pltpu.sync_copy(vals_v.at[slot], acc_sh.at[idx_v.at[ck]], add=True)
            return carry

        lax.fori_loop(0, n_chunks, body, 0)
        plsc.subcore_barrier()
        pltpu.sync_copy(acc_sh.at[pl.ds(tbase, rows_per_tile)],
                        out_hbm.at[cid, pl.ds(tbase, rows_per_tile)])

    return k


def _rbf_from_d2(d2):
    dist = jnp.sqrt(d2 + 1e-8)
    centers = lax.broadcasted_iota(jnp.int32, (1, NB), 1).astype(jnp.float32) * (
        CUTOFF / (NB - 1))
    g = jnp.exp(-((dist - centers) ** 2) / 0.5)
    fc = 0.5 * (jnp.cos(jnp.pi * jnp.clip(dist, 0.0, CUTOFF) / CUTOFF) + 1.0)
    return dist, g * fc


def _edgemsg_body(d2_ref, vs_ref, nfs_ref, w_ref, msg_ref):
    dist, rbf = _rbf_from_d2(d2_ref[...])
    edge_w = jnp.dot(rbf, w_ref[...], preferred_element_type=jnp.float32)
    gate = 1.0 + vs_ref[...] / (3.0 * dist)
    msg_ref[...] = nfs_ref[...] * edge_w * gate


def _tc_edgemsg(d2, vs, nfs, w, be=2000):
    grid = (E // be,)
    return pl.pallas_call(
        _edgemsg_body,
        grid=grid,
        in_specs=[
            pl.BlockSpec((be, 1), lambda b: (b, 0)),
            pl.BlockSpec((be, 1), lambda b: (b, 0)),
            pl.BlockSpec((be, D), lambda b: (b, 0)),
            pl.BlockSpec((NB, D), lambda b: (0, 0)),
        ],
        out_specs=pl.BlockSpec((be, D), lambda b: (b, 0)),
        out_shape=jax.ShapeDtypeStruct((E, D), jnp.float32),
    )(d2, vs, nfs, w)


def _update_body(nf_ref, a0_ref, a1_ref, ws_ref, wm_ref, out_ref):
    agg = a0_ref[...] + a1_ref[...]
    h = (jnp.dot(nf_ref[...], ws_ref[...], preferred_element_type=jnp.float32)
         + jnp.dot(agg, wm_ref[...], preferred_element_type=jnp.float32))
    out_ref[...] = h * jax.nn.sigmoid(h)


def _update_ext_body(nf_ref, a0_ref, a1_ref, ws_ref, wm_ref, wii_ref, wij_ref,
                     out_ref, g_ref, fii_ref):
    agg = a0_ref[...] + a1_ref[...]
    h = (jnp.dot(nf_ref[...], ws_ref[...], preferred_element_type=jnp.float32)
         + jnp.dot(agg, wm_ref[...], preferred_element_type=jnp.float32))
    nf = h * jax.nn.sigmoid(h)
    out_ref[...] = nf
    g_ref[...] = jnp.dot(nf, wij_ref[...], preferred_element_type=jnp.float32)
    t = jnp.dot(nf, wii_ref[...], preferred_element_type=jnp.float32)
    fii_ref[...] = t * jax.nn.sigmoid(t)


def _tc_update(nf, a0, a1, ws, wm, bn=2000):
    grid = (N // bn,)
    blk = pl.BlockSpec((bn, D), lambda b: (b, 0))
    wblk = pl.BlockSpec((D, D), lambda b: (0, 0))
    return pl.pallas_call(
        _update_body, grid=grid,
        in_specs=[blk, blk, blk, wblk, wblk],
        out_specs=blk,
        out_shape=jax.ShapeDtypeStruct((N, D), jnp.float32),
    )(nf, a0, a1, ws, wm)


def _tc_update_ext(nf, a0, a1, ws, wm, wii, wij, bn=2000):
    grid = (N // bn,)
    blk = pl.BlockSpec((bn, D), lambda b: (b, 0))
    wblk = pl.BlockSpec((D, D), lambda b: (0, 0))
    sd = jax.ShapeDtypeStruct((N, D), jnp.float32)
    return pl.pallas_call(
        _update_ext_body, grid=grid,
        in_specs=[blk, blk, blk, wblk, wblk, wblk, wblk],
        out_specs=(blk, blk, blk),
        out_shape=(sd, sd, sd),
    )(nf, a0, a1, ws, wm, wii, wij)


def _offdiag_body(d2_ref, gs_ref, gd_ref, wrbf_ref, wout_ref, prev_ref,
                  out_ref):
    _, rbf = _rbf_from_d2(d2_ref[...])
    ew = jnp.dot(rbf, wrbf_ref[...], preferred_element_type=jnp.float32)
    h = gs_ref[...] + gd_ref[...]
    h = h * jax.nn.sigmoid(h) * ew
    out_ref[...] = prev_ref[...] + jnp.dot(
        h, wout_ref[...], preferred_element_type=jnp.float32)


def _tc_offdiag(d2f, gs, gd, wrbf, wout, prev, be=2000):
    grid = (E // be,)
    return pl.pallas_call(
        _offdiag_body, grid=grid,
        in_specs=[
            pl.BlockSpec((be, 1), lambda b: (b, 0)),
            pl.BlockSpec((be, D), lambda b: (b, 0)),
            pl.BlockSpec((be, D), lambda b: (b, 0)),
            pl.BlockSpec((NB, D), lambda b: (0, 0)),
            pl.BlockSpec((D, OUT), lambda b: (0, 0)),
            pl.BlockSpec((be, OUT), lambda b: (b, 0)),
        ],
        out_specs=pl.BlockSpec((be, OUT), lambda b: (b, 0)),
        out_shape=jax.ShapeDtypeStruct((E, OUT), jnp.float32),
    )(d2f, gs, gd, wrbf, wout, prev)


def _diag_body(f0_ref, f1_ref, n0_ref, w_ref, out_ref):
    s = f0_ref[...] + f1_ref[...] + n0_ref[...]
    out_ref[...] = jnp.dot(s, w_ref[...], preferred_element_type=jnp.float32)


def _tc_diag(f0, f1, n0, w, bn=2000):
    grid = (N // bn,)
    blk = pl.BlockSpec((bn, D), lambda b: (b, 0))
    return pl.pallas_call(
        _diag_body, grid=grid,
        in_specs=[blk, blk, blk, pl.BlockSpec((D, OUT), lambda b: (0, 0))],
        out_specs=pl.BlockSpec((bn, OUT), lambda b: (b, 0)),
        out_shape=jax.ShapeDtypeStruct((N, OUT), jnp.float32),
    )(f0, f1, n0, w)


def kernel(at_no, pos, edge_index, edge_index_full, embed_table, conv_Wrbf,
           conv_Wself, conv_Wmsg, trans_Wii, trans_Wrbf, trans_Wij,
           out_Wii, out_Wij):
    f32 = jnp.float32
    src = edge_index[0].astype(jnp.int32)
    dst = edge_index[1].astype(jnp.int32)
    src_f = edge_index_full[0].astype(jnp.int32)
    dst_f = edge_index_full[1].astype(jnp.int32)

    posf = pos.astype(f32)
    zeros_nd = jnp.zeros((NACC, D), f32)
    dst3d = dst.reshape(NW, -1, CH)

    NPAD = NW * CH * 4
    at_pad = jnp.pad(at_no.astype(jnp.int32), (0, NPAD - N)).reshape(NW, -1, CH)
    nf0 = _make_sc_gather(NPAD)(embed_table.astype(f32), at_pad)[:N]

    d2, vs, d2f = _make_sc_geometry()(
        posf[:, 0], posf[:, 1], posf[:, 2], src, dst, src_f, dst_f)
    d2 = d2.reshape(E, 1)
    vs = vs.reshape(E, 1)
    d2f = d2f.reshape(E, 1)

    g_nf = _make_sc_gather(E)
    g_pair = _make_sc_gather(2 * E)
    scat = _make_sc_scatter_add()
    src3d = src.reshape(NW, -1, CH)
    pair_idx = jnp.concatenate([src_f, dst_f]).reshape(NW, -1, CH)

    nf = nf0
    fii = []
    offd = jnp.zeros((E, OUT), f32)
    for i in range(3):
        nfs = g_nf(nf, src3d)
        msg = _tc_edgemsg(d2, vs, nfs, conv_Wrbf[i].astype(f32))
        aggp = scat(msg, dst3d, zeros_nd)
        aggp = aggp[:, :N]
        if i == 0:
            nf = _tc_update(nf, aggp[0], aggp[1],
                            conv_Wself[i].astype(f32), conv_Wmsg[i].astype(f32))
        else:
            j = i - 1
            nf, g, fii_j = _tc_update_ext(
                nf, aggp[0], aggp[1],
                conv_Wself[i].astype(f32), conv_Wmsg[i].astype(f32),
                trans_Wii[j].astype(f32), trans_Wij[j].astype(f32))
            fii.append(fii_j)
            gpair = g_pair(g, pair_idx)
            gs, gd = gpair[:E], gpair[E:]
            offd = _tc_offdiag(d2f, gs, gd, trans_Wrbf[j].astype(f32),
                               out_Wij.astype(f32), offd)

    diag = _tc_diag(fii[0], fii[1], nf0, out_Wii.astype(f32))
    return (diag, offd)


# speedup vs baseline: 2.2306x; 1.1693x over previous
import functools

import jax
import jax.numpy as jnp
from jax import lax
from jax.experimental import pallas as pl
from jax.experimental.pallas import tpu as pltpu
from jax.experimental.pallas import tpu_sc as plsc

N = 10000
E = 320000
D = 128
NB = 32
OUT = 64
CUTOFF = 5.0

NC = 2
NS = 16
NW = NC * NS
CH = 80
NACC = 10240
NPAD = NW * CH * 4

_MESH = dict(core_axis_name="c", subcore_axis_name="s")
_NOLAYOUT = pltpu.CompilerParams(needs_layout_passes=False)


def _mul_rows(dst_ref, a_ref, b_ref, n_rows):
    def row(r, c):
        for c8 in range(D // 16):
            s = pl.ds(c8 * 16, 16)
            dst_ref[r, s] = a_ref[r, s] * b_ref[r, s]
        return c
    lax.fori_loop(0, n_rows, row, 0)


def _add_rows(dst_ref, a_ref, b_ref, n_rows):
    def row(r, c):
        for c8 in range(D // 16):
            s = pl.ds(c8 * 16, 16)
            dst_ref[r, s] = a_ref[r, s] + b_ref[r, s]
        return c
    lax.fori_loop(0, n_rows, row, 0)


def _make_sc_geo_embed():
    per_w = E // NW
    GCH = 2000
    g_chunks = per_w // GCH
    iters = GCH // 16
    e_chunks = NPAD // (NW * CH)

    sd = jax.ShapeDtypeStruct((E,), jnp.float32)

    @functools.partial(
        pl.kernel,
        out_type=(sd, sd, sd, jax.ShapeDtypeStruct((NPAD, D), jnp.float32)),
        mesh=plsc.VectorSubcoreMesh(**_MESH),
        compiler_params=_NOLAYOUT,
        scratch_types=[
            pltpu.VMEM((N,), jnp.float32),
            pltpu.VMEM((N,), jnp.float32),
            pltpu.VMEM((N,), jnp.float32),
            pltpu.VMEM((GCH,), jnp.int32),
            pltpu.VMEM((GCH,), jnp.int32),
            pltpu.VMEM((GCH,), jnp.float32),
            pltpu.VMEM((GCH,), jnp.float32),
            pltpu.VMEM((e_chunks, CH), jnp.int32),
            pltpu.VMEM((CH, D), jnp.float32),
            pltpu.SemaphoreType.DMA,
        ],
    )
    def k(px_h, py_h, pz_h, src_h, dst_h, srcf_h, dstf_h, emb_h, at_h,
          d2_h, vs_h, d2f_h, nf0_h,
          px, py, pz, si, di, d2v, vsv, eidx, ebuf, sem):
        wid = lax.axis_index("s") * NC + lax.axis_index("c")
        base = pl.multiple_of(wid * per_w, 8)

        ebase = pl.multiple_of(wid * e_chunks * CH, 8)
        pltpu.sync_copy(at_h.at[wid], eidx)
        for ck in range(e_chunks):
            pltpu.async_copy(emb_h.at[eidx.at[ck]], ebuf, sem).wait()
            pltpu.sync_copy(ebuf, nf0_h.at[pl.ds(ebase + ck * CH, CH)])

        pltpu.sync_copy(px_h, px)
        pltpu.sync_copy(py_h, py)
        pltpu.sync_copy(pz_h, pz)

        def run(src_ref, dst_ref, out_d2, out_vs):
            def chunk(gc, carry):
                cbase = pl.multiple_of(base + gc * GCH, 8)
                pltpu.sync_copy(src_ref.at[pl.ds(cbase, GCH)], si)
                pltpu.sync_copy(dst_ref.at[pl.ds(cbase, GCH)], di)

                def body(i, c2):
                    off = pl.multiple_of(i * 16, 8)
                    s16 = si[pl.ds(off, 16)]
                    d16 = di[pl.ds(off, 16)]
                    vx = plsc.load_gather(px, [d16]) - plsc.load_gather(px, [s16])
                    vy = plsc.load_gather(py, [d16]) - plsc.load_gather(py, [s16])
                    vz = plsc.load_gather(pz, [d16]) - plsc.load_gather(pz, [s16])
                    d2v[pl.ds(off, 16)] = vx * vx + vy * vy + vz * vz
                    if out_vs is not None:
                        vsv[pl.ds(off, 16)] = vx + vy + vz
                    return c2
                lax.fori_loop(0, iters, body, 0)
                pltpu.sync_copy(d2v, out_d2.at[pl.ds(cbase, GCH)])
                if out_vs is not None:
                    pltpu.sync_copy(vsv, out_vs.at[pl.ds(cbase, GCH)])
                return carry
            lax.fori_loop(0, g_chunks, chunk, 0)

        run(src_h, dst_h, d2_h, vs_h)
        run(srcf_h, dstf_h, d2f_h, None)

    return k


SBN = 5


def _make_sc_conv():
    per_w = E // NW
    n_chunks = per_w // CH
    rows_per_tile = NACC // NS

    @functools.partial(
        pl.kernel,
        out_type=jax.ShapeDtypeStruct((NC, NACC, D), jnp.float32),
        mesh=plsc.VectorSubcoreMesh(**_MESH),
        scratch_types=[
            pltpu.VMEM_SHARED((NACC, D), jnp.float32),
            pltpu.VMEM((2, CH, D), jnp.float32),
            pltpu.VMEM((2, CH, D), jnp.float32),
            pltpu.VMEM((2, SBN, CH), jnp.int32),
            pltpu.VMEM((2, SBN, CH), jnp.int32),
            pltpu.SemaphoreType.DMA,
            pltpu.SemaphoreType.DMA,
            pltpu.SemaphoreType.DMA,
        ],
    )
    def k(nf_hbm, src_hbm, dst_hbm, ew_hbm, zeros_hbm, out_hbm,
          acc_sh, nfb, ewb, sidx, didx, gsem, esem, ssem):
        sid = lax.axis_index("s")
        cid = lax.axis_index("c")
        wid = sid * NC + cid
        base = pl.multiple_of(wid * per_w, 8)
        tbase = pl.multiple_of(sid * rows_per_tile, 8)
        pltpu.sync_copy(zeros_hbm.at[pl.ds(tbase, rows_per_tile)],
                        acc_sh.at[pl.ds(tbase, rows_per_tile)])

        pltpu.sync_copy(src_hbm.at[wid, 0], sidx.at[0])
        pltpu.sync_copy(dst_hbm.at[wid, 0], didx.at[0])
        plsc.subcore_barrier()
        pltpu.async_copy(nf_hbm.at[sidx.at[0, 0]], nfb.at[0], gsem)
        pltpu.async_copy(ew_hbm.at[pl.ds(base, CH)], ewb.at[0], esem)

        def body(ck, carry):
            slot = lax.rem(ck, 2)
            nxt = lax.rem(ck + 1, 2)
            nb = (ck + 1) // SBN
            nbs = lax.rem(nb, 2)

            @pl.when(ck >= 1)
            def _():
                pltpu.make_async_copy(
                    nfb.at[nxt], acc_sh.at[pl.ds(0, CH)], ssem).wait()

            @pl.when(jnp.logical_and(lax.rem(ck + 1, SBN) == 0,
                                     ck + 1 < n_chunks))
            def _():
                pltpu.sync_copy(src_hbm.at[wid, nb], sidx.at[nbs])
                pltpu.sync_copy(dst_hbm.at[wid, nb], didx.at[nbs])

            @pl.when(ck + 1 < n_chunks)
            def _():
                pltpu.async_copy(
                    nf_hbm.at[sidx.at[nbs, lax.rem(ck + 1, SBN)]],
                    nfb.at[nxt], gsem)
                pltpu.async_copy(ew_hbm.at[pl.ds(base + (ck + 1) * CH, CH)],
                                 ewb.at[nxt], esem)

            pltpu.make_async_copy(nf_hbm.at[sidx.at[0, 0]], nfb.at[slot],
                                  gsem).wait()
            pltpu.make_async_copy(ew_hbm.at[pl.ds(base, CH)], ewb.at[slot],
                                  esem).wait()
            _mul_rows(nfb.at[slot], nfb.at[slot], ewb.at[slot], CH)
            pltpu.make_async_copy(
                nfb.at[slot],
                acc_sh.at[didx.at[lax.rem(ck // SBN, 2), lax.rem(ck, SBN)]],
                ssem).start(add=True)
            return carry

        lax.fori_loop(0, n_chunks, body, 0)
        pltpu.make_async_copy(nfb.at[0], acc_sh.at[pl.ds(0, CH)], ssem).wait()
        plsc.subcore_barrier()
        pltpu.sync_copy(acc_sh.at[pl.ds(tbase, rows_per_tile)],
                        out_hbm.at[cid, pl.ds(tbase, rows_per_tile)])

    return k


def _make_sc_pair_add():
    per_w = E // NW
    n_chunks = per_w // CH

    @functools.partial(
        pl.kernel,
        out_type=jax.ShapeDtypeStruct((E, D), jnp.float32),
        mesh=plsc.VectorSubcoreMesh(**_MESH),
        scratch_types=[
            pltpu.VMEM((2, CH, D), jnp.float32),
            pltpu.VMEM((2, CH, D), jnp.float32),
            pltpu.VMEM((n_chunks, CH), jnp.int32),
            pltpu.VMEM((n_chunks, CH), jnp.int32),
            pltpu.SemaphoreType.DMA,
            pltpu.SemaphoreType.DMA,
            pltpu.SemaphoreType.DMA,
        ],
    )
    def k(g_hbm, src_hbm, dst_hbm, out_hbm, sb, db, sidx, didx,
          s_sem, d_sem, osem):
        wid = lax.axis_index("s") * NC + lax.axis_index("c")
        base = pl.multiple_of(wid * per_w, 8)
        pltpu.sync_copy(src_hbm.at[wid], sidx)
        pltpu.sync_copy(dst_hbm.at[wid], didx)

        pltpu.async_copy(g_hbm.at[sidx.at[0]], sb.at[0], s_sem)
        pltpu.async_copy(g_hbm.at[didx.at[0]], db.at[0], d_sem)

        def body(ck, carry):
            slot = lax.rem(ck, 2)
            nxt = lax.rem(ck + 1, 2)

            @pl.when(jnp.logical_and(ck >= 1, ck + 1 < n_chunks))
            def _():
                pltpu.make_async_copy(
                    sb.at[nxt], out_hbm.at[pl.ds(base, CH)], osem).wait()

            @pl.when(ck + 1 < n_chunks)
            def _():
                pltpu.async_copy(g_hbm.at[sidx.at[ck + 1]], sb.at[nxt], s_sem)
                pltpu.async_copy(g_hbm.at[didx.at[ck + 1]], db.at[nxt], d_sem)

            pltpu.make_async_copy(g_hbm.at[sidx.at[ck]], sb.at[slot],
                                  s_sem).wait()
            pltpu.make_async_copy(g_hbm.at[didx.at[ck]], db.at[slot],
                                  d_sem).wait()
            _add_rows(sb.at[slot], sb.at[slot], db.at[slot], CH)
            pltpu.async_copy(sb.at[slot],
                             out_hbm.at[pl.ds(base + ck * CH, CH)], osem)
            return carry

        lax.fori_loop(0, n_chunks, body, 0)
        pltpu.make_async_copy(sb.at[0], out_hbm.at[pl.ds(base, CH)],
                              osem).wait()
        pltpu.make_async_copy(sb.at[1], out_hbm.at[pl.ds(base, CH)],
                              osem).wait()

    return k


def _rbf_from_d2(d2):
    dist = jnp.sqrt(d2 + 1e-8)
    centers = lax.broadcasted_iota(jnp.int32, (1, NB), 1).astype(jnp.float32) * (
        CUTOFF / (NB - 1))
    g = jnp.exp(-((dist - centers) ** 2) / 0.5)
    fc = 0.5 * (jnp.cos(jnp.pi * jnp.clip(dist, 0.0, CUTOFF) / CUTOFF) + 1.0)
    return dist, g * fc


def _ew_body(d2_ref, vs_ref, w_ref, ew0_ref, ew1_ref, ew2_ref):
    dist, rbf = _rbf_from_d2(d2_ref[...])
    gate = 1.0 + vs_ref[...] / (3.0 * dist)
    rbf = rbf * gate
    ew0_ref[...] = jnp.dot(rbf, w_ref[0], preferred_element_type=jnp.float32)
    ew1_ref[...] = jnp.dot(rbf, w_ref[1], preferred_element_type=jnp.float32)
    ew2_ref[...] = jnp.dot(rbf, w_ref[2], preferred_element_type=jnp.float32)


def _tc_ew_all(d2, vs, w3, be=2000):
    grid = (E // be,)
    eblk = pl.BlockSpec((be, D), lambda b: (b, 0))
    sd = jax.ShapeDtypeStruct((E, D), jnp.float32)
    return pl.pallas_call(
        _ew_body,
        grid=grid,
        in_specs=[
            pl.BlockSpec((be, 1), lambda b: (b, 0)),
            pl.BlockSpec((be, 1), lambda b: (b, 0)),
            pl.BlockSpec((3, NB, D), lambda b: (0, 0, 0)),
        ],
        out_specs=(eblk, eblk, eblk),
        out_shape=(sd, sd, sd),
    )(d2, vs, w3)


def _update_body(nf_ref, agg_ref, ws_ref, wm_ref, out_ref):
    agg = agg_ref[0] + agg_ref[1]
    h = (jnp.dot(nf_ref[...], ws_ref[...], preferred_element_type=jnp.float32)
         + jnp.dot(agg, wm_ref[...], preferred_element_type=jnp.float32))
    out_ref[...] = h * jax.nn.sigmoid(h)


def _update_ext_body(nf_ref, agg_ref, ws_ref, wm_ref, wii_ref, wij_ref,
                     out_ref, g_ref, fii_ref):
    agg = agg_ref[0] + agg_ref[1]
    h = (jnp.dot(nf_ref[...], ws_ref[...], preferred_element_type=jnp.float32)
         + jnp.dot(agg, wm_ref[...], preferred_element_type=jnp.float32))
    nf = h * jax.nn.sigmoid(h)
    out_ref[...] = nf
    g_ref[...] = jnp.dot(nf, wij_ref[...], preferred_element_type=jnp.float32)
    t = jnp.dot(nf, wii_ref[...], preferred_element_type=jnp.float32)
    fii_ref[...] = t * jax.nn.sigmoid(t)


def _tc_update(nf, aggp, ws, wm, bn=2000):
    grid = (N // bn,)
    blk = pl.BlockSpec((bn, D), lambda b: (b, 0))
    ablk = pl.BlockSpec((NC, bn, D), lambda b: (0, b, 0))
    wblk = pl.BlockSpec((D, D), lambda b: (0, 0))
    return pl.pallas_call(
        _update_body, grid=grid,
        in_specs=[blk, ablk, wblk, wblk],
        out_specs=blk,
        out_shape=jax.ShapeDtypeStruct((N, D), jnp.float32),
    )(nf, aggp, ws, wm)


def _tc_update_ext(nf, aggp, ws, wm, wii, wij, bn=2000):
    grid = (N // bn,)
    blk = pl.BlockSpec((bn, D), lambda b: (b, 0))
    ablk = pl.BlockSpec((NC, bn, D), lambda b: (0, b, 0))
    wblk = pl.BlockSpec((D, D), lambda b: (0, 0))
    sd = jax.ShapeDtypeStruct((N, D), jnp.float32)
    return pl.pallas_call(
        _update_ext_body, grid=grid,
        in_specs=[blk, ablk, wblk, wblk, wblk, wblk],
        out_specs=(blk, blk, blk),
        out_shape=(sd, sd, sd),
    )(nf, aggp, ws, wm, wii, wij)


def _offdiag_body0(d2_ref, h_ref, wrbf_ref, wout_ref, out_ref):
    _, rbf = _rbf_from_d2(d2_ref[...])
    ew = jnp.dot(rbf, wrbf_ref[...], preferred_element_type=jnp.float32)
    h = h_ref[...]
    h = h * jax.nn.sigmoid(h) * ew
    out_ref[...] = jnp.dot(h, wout_ref[...], preferred_element_type=jnp.float32)


def _offdiag_body1(d2_ref, h_ref, wrbf_ref, wout_ref, prev_ref, out_ref):
    _, rbf = _rbf_from_d2(d2_ref[...])
    ew = jnp.dot(rbf, wrbf_ref[...], preferred_element_type=jnp.float32)
    h = h_ref[...]
    h = h * jax.nn.sigmoid(h) * ew
    out_ref[...] = prev_ref[...] + jnp.dot(
        h, wout_ref[...], preferred_element_type=jnp.float32)


def _tc_offdiag(d2f, h, wrbf, wout, prev=None, be=2000):
    grid = (E // be,)
    specs = [
        pl.BlockSpec((be, 1), lambda b: (b, 0)),
        pl.BlockSpec((be, D), lambda b: (b, 0)),
        pl.BlockSpec((NB, D), lambda b: (0, 0)),
        pl.BlockSpec((D, OUT), lambda b: (0, 0)),
    ]
    args = [d2f, h, wrbf, wout]
    body = _offdiag_body0
    if prev is not None:
        specs.append(pl.BlockSpec((be, OUT), lambda b: (b, 0)))
        args.append(prev)
        body = _offdiag_body1
    return pl.pallas_call(
        body, grid=grid,
        in_specs=specs,
        out_specs=pl.BlockSpec((be, OUT), lambda b: (b, 0)),
        out_shape=jax.ShapeDtypeStruct((E, OUT), jnp.float32),
    )(*args)


def _diag_body(f0_ref, f1_ref, n0_ref, w_ref, out_ref):
    s = f0_ref[...] + f1_ref[...] + n0_ref[...]
    out_ref[...] = jnp.dot(s, w_ref[...], preferred_element_type=jnp.float32)


def _tc_diag(f0, f1, n0, w, bn=2000):
    grid = (N // bn,)
    blk = pl.BlockSpec((bn, D), lambda b: (b, 0))
    return pl.pallas_call(
        _diag_body, grid=grid,
        in_specs=[blk, blk, blk, pl.BlockSpec((D, OUT), lambda b: (0, 0))],
        out_specs=pl.BlockSpec((bn, OUT), lambda b: (b, 0)),
        out_shape=jax.ShapeDtypeStruct((N, OUT), jnp.float32),
    )(f0, f1, n0, w)


def kernel(at_no, pos, edge_index, edge_index_full, embed_table, conv_Wrbf,
           conv_Wself, conv_Wmsg, trans_Wii, trans_Wrbf, trans_Wij,
           out_Wii, out_Wij):
    f32 = jnp.float32
    src = edge_index[0].astype(jnp.int32)
    dst = edge_index[1].astype(jnp.int32)
    src_f = edge_index_full[0].astype(jnp.int32)
    dst_f = edge_index_full[1].astype(jnp.int32)

    posf = pos.astype(f32)
    zeros_nd = jnp.zeros((NACC, D), f32)
    src4d = src.reshape(NW, -1, SBN, CH)
    dst4d = dst.reshape(NW, -1, SBN, CH)
    srcf3d = src_f.reshape(NW, -1, CH)
    dstf3d = dst_f.reshape(NW, -1, CH)
    at3d = jnp.pad(at_no.astype(jnp.int32), (0, NPAD - N)).reshape(NW, -1, CH)

    d2, vs, d2f, nf0p = _make_sc_geo_embed()(
        posf[:, 0], posf[:, 1], posf[:, 2], src, dst, src_f, dst_f,
        embed_table.astype(f32), at3d)
    nf0 = nf0p[:N]
    d2 = d2.reshape(E, 1)
    vs = vs.reshape(E, 1)
    d2f = d2f.reshape(E, 1)

    ews = _tc_ew_all(d2, vs, conv_Wrbf.astype(f32))
    pair_add = _make_sc_pair_add()
    conv = _make_sc_conv()

    nf = nf0
    fii = []
    offd = None
    for i in range(3):
        aggp = conv(nf, src4d, dst4d, ews[i], zeros_nd)
        if i == 0:
            nf = _tc_update(nf, aggp, conv_Wself[i].astype(f32),
                            conv_Wmsg[i].astype(f32))
        else:
            j = i - 1
            nf, g, fii_j = _tc_update_ext(
                nf, aggp, conv_Wself[i].astype(f32), conv_Wmsg[i].astype(f32),
                trans_Wii[j].astype(f32), trans_Wij[j].astype(f32))
            fii.append(fii_j)
            h = pair_add(g, srcf3d, dstf3d)
            offd = _tc_offdiag(d2f, h, trans_Wrbf[j].astype(f32),
                               out_Wij.astype(f32), offd)

    diag = _tc_diag(fii[0], fii[1], nf0, out_Wii.astype(f32))
    return (diag, offd)


# speedup vs baseline: 3.0479x; 1.3664x over previous
import functools

import jax
import jax.numpy as jnp
from jax import lax
from jax.experimental import pallas as pl
from jax.experimental.pallas import tpu as pltpu
from jax.experimental.pallas import tpu_sc as plsc

N = 10000
E = 320000
D = 128
NB = 32
OUT = 64
CUTOFF = 5.0

NC = 2
NS = 16
NW = NC * NS
CH = 80
NACC = 10240
NPAD = NW * CH * 4

_MESH = dict(core_axis_name="c", subcore_axis_name="s")
_NOLAYOUT = pltpu.CompilerParams(needs_layout_passes=False)


def _mul_rows(dst_ref, a_ref, b_ref, n_rows):
    @plsc.parallel_loop(0, n_rows, 1, unroll=4)
    def row(r):
        for c8 in range(D // 16):
            s = pl.ds(c8 * 16, 16)
            dst_ref[r, s] = a_ref[r, s] * b_ref[r, s]


def _add_rows(dst_ref, a_ref, b_ref, n_rows):
    @plsc.parallel_loop(0, n_rows, 1, unroll=4)
    def row(r):
        for c8 in range(D // 16):
            s = pl.ds(c8 * 16, 16)
            dst_ref[r, s] = a_ref[r, s] + b_ref[r, s]


def _make_sc_geo_embed():
    per_w = E // NW
    GCH = 2000
    g_chunks = per_w // GCH
    iters = GCH // 16
    e_chunks = NPAD // (NW * CH)

    sd = jax.ShapeDtypeStruct((E,), jnp.float32)

    @functools.partial(
        pl.kernel,
        out_type=(sd, sd, sd, jax.ShapeDtypeStruct((NPAD, D), jnp.float32)),
        mesh=plsc.VectorSubcoreMesh(**_MESH),
        compiler_params=_NOLAYOUT,
        scratch_types=[
            pltpu.VMEM((N,), jnp.float32),
            pltpu.VMEM((N,), jnp.float32),
            pltpu.VMEM((N,), jnp.float32),
            pltpu.VMEM((GCH,), jnp.int32),
            pltpu.VMEM((GCH,), jnp.int32),
            pltpu.VMEM((GCH,), jnp.float32),
            pltpu.VMEM((GCH,), jnp.float32),
            pltpu.VMEM((e_chunks, CH), jnp.int32),
            pltpu.VMEM((CH, D), jnp.float32),
            pltpu.SemaphoreType.DMA,
        ],
    )
    def k(px_h, py_h, pz_h, src_h, dst_h, srcf_h, dstf_h, emb_h, at_h,
          d2_h, vs_h, d2f_h, nf0_h,
          px, py, pz, si, di, d2v, vsv, eidx, ebuf, sem):
        wid = lax.axis_index("s") * NC + lax.axis_index("c")
        base = pl.multiple_of(wid * per_w, 8)

        ebase = pl.multiple_of(wid * e_chunks * CH, 8)
        pltpu.sync_copy(at_h.at[wid], eidx)
        for ck in range(e_chunks):
            pltpu.async_copy(emb_h.at[eidx.at[ck]], ebuf, sem).wait()
            pltpu.sync_copy(ebuf, nf0_h.at[pl.ds(ebase + ck * CH, CH)])

        pltpu.sync_copy(px_h, px)
        pltpu.sync_copy(py_h, py)
        pltpu.sync_copy(pz_h, pz)

        def run(src_ref, dst_ref, out_d2, out_vs):
            def chunk(gc, carry):
                cbase = pl.multiple_of(base + gc * GCH, 8)
                pltpu.sync_copy(src_ref.at[pl.ds(cbase, GCH)], si)
                pltpu.sync_copy(dst_ref.at[pl.ds(cbase, GCH)], di)

                @plsc.parallel_loop(0, iters, 1, unroll=4)
                def body(i):
                    off = pl.multiple_of(i * 16, 8)
                    s16 = si[pl.ds(off, 16)]
                    d16 = di[pl.ds(off, 16)]
                    vx = plsc.load_gather(px, [d16]) - plsc.load_gather(px, [s16])
                    vy = plsc.load_gather(py, [d16]) - plsc.load_gather(py, [s16])
                    vz = plsc.load_gather(pz, [d16]) - plsc.load_gather(pz, [s16])
                    d2v[pl.ds(off, 16)] = vx * vx + vy * vy + vz * vz
                    if out_vs is not None:
                        vsv[pl.ds(off, 16)] = vx + vy + vz
                pltpu.sync_copy(d2v, out_d2.at[pl.ds(cbase, GCH)])
                if out_vs is not None:
                    pltpu.sync_copy(vsv, out_vs.at[pl.ds(cbase, GCH)])
                return carry
            lax.fori_loop(0, g_chunks, chunk, 0)

        run(src_h, dst_h, d2_h, vs_h)
        run(srcf_h, dstf_h, d2f_h, None)

    return k


SBN = 5


def _make_sc_conv():
    per_w = E // NW
    n_chunks = per_w // CH
    rows_per_tile = NACC // NS

    @functools.partial(
        pl.kernel,
        out_type=jax.ShapeDtypeStruct((NC, NACC, D), jnp.float32),
        mesh=plsc.VectorSubcoreMesh(**_MESH),
        scratch_types=[
            pltpu.VMEM_SHARED((NACC, D), jnp.float32),
            pltpu.VMEM((2, CH, D), jnp.float32),
            pltpu.VMEM((2, CH, D), jnp.float32),
            pltpu.VMEM((2, SBN, CH), jnp.int32),
            pltpu.VMEM((2, SBN, CH), jnp.int32),
            pltpu.SemaphoreType.DMA,
            pltpu.SemaphoreType.DMA,
            pltpu.SemaphoreType.DMA,
        ],
    )
    def k(nf_hbm, src_hbm, dst_hbm, ew_hbm, zeros_hbm, out_hbm,
          acc_sh, nfb, ewb, sidx, didx, gsem, esem, ssem):
        sid = lax.axis_index("s")
        cid = lax.axis_index("c")
        wid = sid * NC + cid
        base = pl.multiple_of(wid * per_w, 8)
        tbase = pl.multiple_of(sid * rows_per_tile, 8)
        pltpu.sync_copy(zeros_hbm.at[pl.ds(tbase, rows_per_tile)],
                        acc_sh.at[pl.ds(tbase, rows_per_tile)])

        pltpu.sync_copy(src_hbm.at[wid, 0], sidx.at[0])
        pltpu.sync_copy(dst_hbm.at[wid, 0], didx.at[0])
        plsc.subcore_barrier()
        pltpu.async_copy(nf_hbm.at[sidx.at[0, 0]], nfb.at[0], gsem)
        pltpu.async_copy(ew_hbm.at[pl.ds(base, CH)], ewb.at[0], esem)

        def body(ck, carry):
            slot = lax.rem(ck, 2)
            nxt = lax.rem(ck + 1, 2)
            nb = (ck + 1) // SBN
            nbs = lax.rem(nb, 2)

            @pl.when(ck >= 1)
            def _():
                pltpu.make_async_copy(
                    nfb.at[nxt], acc_sh.at[pl.ds(0, CH)], ssem).wait()

            @pl.when(jnp.logical_and(lax.rem(ck + 1, SBN) == 0,
                                     ck + 1 < n_chunks))
            def _():
                pltpu.sync_copy(src_hbm.at[wid, nb], sidx.at[nbs])
                pltpu.sync_copy(dst_hbm.at[wid, nb], didx.at[nbs])

            @pl.when(ck + 1 < n_chunks)
            def _():
                pltpu.async_copy(
                    nf_hbm.at[sidx.at[nbs, lax.rem(ck + 1, SBN)]],
                    nfb.at[nxt], gsem)
                pltpu.async_copy(ew_hbm.at[pl.ds(base + (ck + 1) * CH, CH)],
                                 ewb.at[nxt], esem)

            pltpu.make_async_copy(nf_hbm.at[sidx.at[0, 0]], nfb.at[slot],
                                  gsem).wait()
            pltpu.make_async_copy(ew_hbm.at[pl.ds(base, CH)], ewb.at[slot],
                                  esem).wait()
            _mul_rows(nfb.at[slot], nfb.at[slot], ewb.at[slot], CH)
            pltpu.make_async_copy(
                nfb.at[slot],
                acc_sh.at[didx.at[lax.rem(ck // SBN, 2), lax.rem(ck, SBN)]],
                ssem).start(add=True)
            return carry

        lax.fori_loop(0, n_chunks, body, 0)
        pltpu.make_async_copy(nfb.at[0], acc_sh.at[pl.ds(0, CH)], ssem).wait()
        plsc.subcore_barrier()
        pltpu.sync_copy(acc_sh.at[pl.ds(tbase, rows_per_tile)],
                        out_hbm.at[cid, pl.ds(tbase, rows_per_tile)])

    return k


def _make_sc_pair_add():
    per_w = E // NW
    n_chunks = per_w // CH

    @functools.partial(
        pl.kernel,
        out_type=jax.ShapeDtypeStruct((E, D), jnp.float32),
        mesh=plsc.VectorSubcoreMesh(**_MESH),
        scratch_types=[
            pltpu.VMEM((2, CH, D), jnp.float32),
            pltpu.VMEM((2, CH, D), jnp.float32),
            pltpu.VMEM((n_chunks, CH), jnp.int32),
            pltpu.VMEM((n_chunks, CH), jnp.int32),
            pltpu.SemaphoreType.DMA,
            pltpu.SemaphoreType.DMA,
            pltpu.SemaphoreType.DMA,
        ],
    )
    def k(g_hbm, src_hbm, dst_hbm, out_hbm, sb, db, sidx, didx,
          s_sem, d_sem, osem):
        wid = lax.axis_index("s") * NC + lax.axis_index("c")
        base = pl.multiple_of(wid * per_w, 8)
        pltpu.sync_copy(src_hbm.at[wid], sidx)
        pltpu.sync_copy(dst_hbm.at[wid], didx)

        pltpu.async_copy(g_hbm.at[sidx.at[0]], sb.at[0], s_sem)
        pltpu.async_copy(g_hbm.at[didx.at[0]], db.at[0], d_sem)

        def body(ck, carry):
            slot = lax.rem(ck, 2)
            nxt = lax.rem(ck + 1, 2)

            @pl.when(jnp.logical_and(ck >= 1, ck + 1 < n_chunks))
            def _():
                pltpu.make_async_copy(
                    sb.at[nxt], out_hbm.at[pl.ds(base, CH)], osem).wait()

            @pl.when(ck + 1 < n_chunks)
            def _():
                pltpu.async_copy(g_hbm.at[sidx.at[ck + 1]], sb.at[nxt], s_sem)
                pltpu.async_copy(g_hbm.at[didx.at[ck + 1]], db.at[nxt], d_sem)

            pltpu.make_async_copy(g_hbm.at[sidx.at[ck]], sb.at[slot],
                                  s_sem).wait()
            pltpu.make_async_copy(g_hbm.at[didx.at[ck]], db.at[slot],
                                  d_sem).wait()
            _add_rows(sb.at[slot], sb.at[slot], db.at[slot], CH)
            pltpu.async_copy(sb.at[slot],
                             out_hbm.at[pl.ds(base + ck * CH, CH)], osem)
            return carry

        lax.fori_loop(0, n_chunks, body, 0)
        pltpu.make_async_copy(sb.at[0], out_hbm.at[pl.ds(base, CH)],
                              osem).wait()
        pltpu.make_async_copy(sb.at[1], out_hbm.at[pl.ds(base, CH)],
                              osem).wait()

    return k


def _rbf_from_d2(d2):
    dist = jnp.sqrt(d2 + 1e-8)
    centers = lax.broadcasted_iota(jnp.int32, (1, NB), 1).astype(jnp.float32) * (
        CUTOFF / (NB - 1))
    g = jnp.exp(-((dist - centers) ** 2) / 0.5)
    fc = 0.5 * (jnp.cos(jnp.pi * jnp.clip(dist, 0.0, CUTOFF) / CUTOFF) + 1.0)
    return dist, g * fc


def _ew_body(d2_ref, vs_ref, w_ref, ew0_ref, ew1_ref, ew2_ref):
    dist, rbf = _rbf_from_d2(d2_ref[...])
    gate = 1.0 + vs_ref[...] / (3.0 * dist)
    rbf = rbf * gate
    ew0_ref[...] = jnp.dot(rbf, w_ref[0], preferred_element_type=jnp.float32)
    ew1_ref[...] = jnp.dot(rbf, w_ref[1], preferred_element_type=jnp.float32)
    ew2_ref[...] = jnp.dot(rbf, w_ref[2], preferred_element_type=jnp.float32)


def _tc_ew_all(d2, vs, w3, be=2000):
    grid = (E // be,)
    eblk = pl.BlockSpec((be, D), lambda b: (b, 0))
    sd = jax.ShapeDtypeStruct((E, D), jnp.float32)
    return pl.pallas_call(
        _ew_body,
        grid=grid,
        in_specs=[
            pl.BlockSpec((be, 1), lambda b: (b, 0)),
            pl.BlockSpec((be, 1), lambda b: (b, 0)),
            pl.BlockSpec((3, NB, D), lambda b: (0, 0, 0)),
        ],
        out_specs=(eblk, eblk, eblk),
        out_shape=(sd, sd, sd),
    )(d2, vs, w3)


def _update_body(nf_ref, agg_ref, ws_ref, wm_ref, out_ref):
    agg = agg_ref[0] + agg_ref[1]
    h = (jnp.dot(nf_ref[...], ws_ref[...], preferred_element_type=jnp.float32)
         + jnp.dot(agg, wm_ref[...], preferred_element_type=jnp.float32))
    out_ref[...] = h * jax.nn.sigmoid(h)


def _update_ext_body(nf_ref, agg_ref, ws_ref, wm_ref, wii_ref, wij_ref,
                     out_ref, g_ref, fii_ref):
    agg = agg_ref[0] + agg_ref[1]
    h = (jnp.dot(nf_ref[...], ws_ref[...], preferred_element_type=jnp.float32)
         + jnp.dot(agg, wm_ref[...], preferred_element_type=jnp.float32))
    nf = h * jax.nn.sigmoid(h)
    out_ref[...] = nf
    g_ref[...] = jnp.dot(nf, wij_ref[...], preferred_element_type=jnp.float32)
    t = jnp.dot(nf, wii_ref[...], preferred_element_type=jnp.float32)
    fii_ref[...] = t * jax.nn.sigmoid(t)


def _tc_update(nf, aggp, ws, wm, bn=2000):
    grid = (N // bn,)
    blk = pl.BlockSpec((bn, D), lambda b: (b, 0))
    ablk = pl.BlockSpec((NC, bn, D), lambda b: (0, b, 0))
    wblk = pl.BlockSpec((D, D), lambda b: (0, 0))
    return pl.pallas_call(
        _update_body, grid=grid,
        in_specs=[blk, ablk, wblk, wblk],
        out_specs=blk,
        out_shape=jax.ShapeDtypeStruct((N, D), jnp.float32),
    )(nf, aggp, ws, wm)


def _tc_update_ext(nf, aggp, ws, wm, wii, wij, bn=2000):
    grid = (N // bn,)
    blk = pl.BlockSpec((bn, D), lambda b: (b, 0))
    ablk = pl.BlockSpec((NC, bn, D), lambda b: (0, b, 0))
    wblk = pl.BlockSpec((D, D), lambda b: (0, 0))
    sd = jax.ShapeDtypeStruct((N, D), jnp.float32)
    return pl.pallas_call(
        _update_ext_body, grid=grid,
        in_specs=[blk, ablk, wblk, wblk, wblk, wblk],
        out_specs=(blk, blk, blk),
        out_shape=(sd, sd, sd),
    )(nf, aggp, ws, wm, wii, wij)


def _offdiag_body0(d2_ref, h_ref, wrbf_ref, wout_ref, out_ref):
    _, rbf = _rbf_from_d2(d2_ref[...])
    ew = jnp.dot(rbf, wrbf_ref[...], preferred_element_type=jnp.float32)
    h = h_ref[...]
    h = h * jax.nn.sigmoid(h) * ew
    out_ref[...] = jnp.dot(h, wout_ref[...], preferred_element_type=jnp.float32)


def _offdiag_body1(d2_ref, h_ref, wrbf_ref, wout_ref, prev_ref, out_ref):
    _, rbf = _rbf_from_d2(d2_ref[...])
    ew = jnp.dot(rbf, wrbf_ref[...], preferred_element_type=jnp.float32)
    h = h_ref[...]
    h = h * jax.nn.sigmoid(h) * ew
    out_ref[...] = prev_ref[...] + jnp.dot(
        h, wout_ref[...], preferred_element_type=jnp.float32)


def _tc_offdiag(d2f, h, wrbf, wout, prev=None, be=2000):
    grid = (E // be,)
    specs = [
        pl.BlockSpec((be, 1), lambda b: (b, 0)),
        pl.BlockSpec((be, D), lambda b: (b, 0)),
        pl.BlockSpec((NB, D), lambda b: (0, 0)),
        pl.BlockSpec((D, OUT), lambda b: (0, 0)),
    ]
    args = [d2f, h, wrbf, wout]
    body = _offdiag_body0
    if prev is not None:
        specs.append(pl.BlockSpec((be, OUT), lambda b: (b, 0)))
        args.append(prev)
        body = _offdiag_body1
    return pl.pallas_call(
        body, grid=grid,
        in_specs=specs,
        out_specs=pl.BlockSpec((be, OUT), lambda b: (b, 0)),
        out_shape=jax.ShapeDtypeStruct((E, OUT), jnp.float32),
    )(*args)


def _diag_body(f0_ref, f1_ref, n0_ref, w_ref, out_ref):
    s = f0_ref[...] + f1_ref[...] + n0_ref[...]
    out_ref[...] = jnp.dot(s, w_ref[...], preferred_element_type=jnp.float32)


def _tc_diag(f0, f1, n0, w, bn=2000):
    grid = (N // bn,)
    blk = pl.BlockSpec((bn, D), lambda b: (b, 0))
    return pl.pallas_call(
        _diag_body, grid=grid,
        in_specs=[blk, blk, blk, pl.BlockSpec((D, OUT), lambda b: (0, 0))],
        out_specs=pl.BlockSpec((bn, OUT), lambda b: (b, 0)),
        out_shape=jax.ShapeDtypeStruct((N, OUT), jnp.float32),
    )(f0, f1, n0, w)


def kernel(at_no, pos, edge_index, edge_index_full, embed_table, conv_Wrbf,
           conv_Wself, conv_Wmsg, trans_Wii, trans_Wrbf, trans_Wij,
           out_Wii, out_Wij):
    f32 = jnp.float32
    src = edge_index[0].astype(jnp.int32)
    dst = edge_index[1].astype(jnp.int32)
    src_f = edge_index_full[0].astype(jnp.int32)
    dst_f = edge_index_full[1].astype(jnp.int32)

    posf = pos.astype(f32)
    zeros_nd = jnp.zeros((NACC, D), f32)
    src4d = src.reshape(NW, -1, SBN, CH)
    dst4d = dst.reshape(NW, -1, SBN, CH)
    srcf3d = src_f.reshape(NW, -1, CH)
    dstf3d = dst_f.reshape(NW, -1, CH)
    at3d = jnp.pad(at_no.astype(jnp.int32), (0, NPAD - N)).reshape(NW, -1, CH)

    d2, vs, d2f, nf0p = _make_sc_geo_embed()(
        posf[:, 0], posf[:, 1], posf[:, 2], src, dst, src_f, dst_f,
        embed_table.astype(f32), at3d)
    nf0 = nf0p[:N]
    d2 = d2.reshape(E, 1)
    vs = vs.reshape(E, 1)
    d2f = d2f.reshape(E, 1)

    ews = _tc_ew_all(d2, vs, conv_Wrbf.astype(f32))
    pair_add = _make_sc_pair_add()
    conv = _make_sc_conv()

    nf = nf0
    fii = []
    offd = None
    for i in range(3):
        aggp = conv(nf, src4d, dst4d, ews[i], zeros_nd)
        if i == 0:
            nf = _tc_update(nf, aggp, conv_Wself[i].astype(f32),
                            conv_Wmsg[i].astype(f32))
        else:
            j = i - 1
            nf, g, fii_j = _tc_update_ext(
                nf, aggp, conv_Wself[i].astype(f32), conv_Wmsg[i].astype(f32),
                trans_Wii[j].astype(f32), trans_Wij[j].astype(f32))
            fii.append(fii_j)
            h = pair_add(g, srcf3d, dstf3d)
            offd = _tc_offdiag(d2f, h, trans_Wrbf[j].astype(f32),
                               out_Wij.astype(f32), offd)

    diag = _tc_diag(fii[0], fii[1], nf0, out_Wii.astype(f32))
    return (diag, offd)


# speedup vs baseline: 4.4003x; 1.4437x over previous
import functools

import jax
import jax.numpy as jnp
from jax import lax
from jax.experimental import pallas as pl
from jax.experimental.pallas import tpu as pltpu
from jax.experimental.pallas import tpu_sc as plsc

N = 10000
E = 320000
D = 128
NB = 32
OUT = 64
CUTOFF = 5.0

NC = 2
NS = 16
NW = NC * NS
CH = 80
NACC = 10240
NPAD = NW * CH * 4

_MESH = dict(core_axis_name="c", subcore_axis_name="s")
_NOLAYOUT = pltpu.CompilerParams(needs_layout_passes=False)


def _mul_rows(dst_ref, a_ref, b_ref, n_rows):
    @plsc.parallel_loop(0, n_rows, 1, unroll=4)
    def row(r):
        for c8 in range(D // 16):
            s = pl.ds(c8 * 16, 16)
            dst_ref[r, s] = a_ref[r, s] * b_ref[r, s]


def _add_rows(dst_ref, a_ref, b_ref, n_rows):
    @plsc.parallel_loop(0, n_rows, 1, unroll=4)
    def row(r):
        for c8 in range(D // 16):
            s = pl.ds(c8 * 16, 16)
            dst_ref[r, s] = a_ref[r, s] + b_ref[r, s]


def _make_sc_geo_embed():
    per_w = E // NW
    GCH = 2000
    g_chunks = per_w // GCH
    iters = GCH // 16
    e_chunks = NPAD // (NW * CH)

    sd = jax.ShapeDtypeStruct((E,), jnp.float32)

    @functools.partial(
        pl.kernel,
        out_type=(sd, sd, sd, jax.ShapeDtypeStruct((NPAD, D), jnp.float32)),
        mesh=plsc.VectorSubcoreMesh(**_MESH),
        compiler_params=_NOLAYOUT,
        scratch_types=[
            pltpu.VMEM((N,), jnp.float32),
            pltpu.VMEM((N,), jnp.float32),
            pltpu.VMEM((N,), jnp.float32),
            pltpu.VMEM((GCH,), jnp.int32),
            pltpu.VMEM((GCH,), jnp.int32),
            pltpu.VMEM((GCH,), jnp.float32),
            pltpu.VMEM((GCH,), jnp.float32),
            pltpu.VMEM((e_chunks, CH), jnp.int32),
            pltpu.VMEM((CH, D), jnp.float32),
            pltpu.SemaphoreType.DMA,
        ],
    )
    def k(px_h, py_h, pz_h, src_h, dst_h, srcf_h, dstf_h, emb_h, at_h,
          d2_h, vs_h, d2f_h, nf0_h,
          px, py, pz, si, di, d2v, vsv, eidx, ebuf, sem):
        wid = lax.axis_index("s") * NC + lax.axis_index("c")
        base = pl.multiple_of(wid * per_w, 8)

        ebase = pl.multiple_of(wid * e_chunks * CH, 8)
        pltpu.sync_copy(at_h.at[wid], eidx)
        for ck in range(e_chunks):
            pltpu.async_copy(emb_h.at[eidx.at[ck]], ebuf, sem).wait()
            pltpu.sync_copy(ebuf, nf0_h.at[pl.ds(ebase + ck * CH, CH)])

        pltpu.sync_copy(px_h, px)
        pltpu.sync_copy(py_h, py)
        pltpu.sync_copy(pz_h, pz)

        def run(src_ref, dst_ref, out_d2, out_vs):
            def chunk(gc, carry):
                cbase = pl.multiple_of(base + gc * GCH, 8)
                pltpu.sync_copy(src_ref.at[pl.ds(cbase, GCH)], si)
                pltpu.sync_copy(dst_ref.at[pl.ds(cbase, GCH)], di)

                @plsc.parallel_loop(0, iters, 1, unroll=4)
                def body(i):
                    off = pl.multiple_of(i * 16, 8)
                    s16 = si[pl.ds(off, 16)]
                    d16 = di[pl.ds(off, 16)]
                    vx = plsc.load_gather(px, [d16]) - plsc.load_gather(px, [s16])
                    vy = plsc.load_gather(py, [d16]) - plsc.load_gather(py, [s16])
                    vz = plsc.load_gather(pz, [d16]) - plsc.load_gather(pz, [s16])
                    d2v[pl.ds(off, 16)] = vx * vx + vy * vy + vz * vz
                    if out_vs is not None:
                        vsv[pl.ds(off, 16)] = vx + vy + vz
                pltpu.sync_copy(d2v, out_d2.at[pl.ds(cbase, GCH)])
                if out_vs is not None:
                    pltpu.sync_copy(vsv, out_vs.at[pl.ds(cbase, GCH)])
                return carry
            lax.fori_loop(0, g_chunks, chunk, 0)

        run(src_h, dst_h, d2_h, vs_h)
        run(srcf_h, dstf_h, d2f_h, None)

    return k


SBN = 5


def _make_sc_conv():
    per_w = E // NW
    n_chunks = per_w // CH
    rows_per_tile = NACC // NS

    @functools.partial(
        pl.kernel,
        out_type=jax.ShapeDtypeStruct((NC, NACC, D), jnp.float32),
        mesh=plsc.VectorSubcoreMesh(**_MESH),
        scratch_types=[
            pltpu.VMEM_SHARED((NACC, D), jnp.float32),
            pltpu.VMEM((2, CH, D), jnp.float32),
            pltpu.VMEM((2, CH, D), jnp.float32),
            pltpu.VMEM((2, SBN, CH), jnp.int32),
            pltpu.VMEM((2, SBN, CH), jnp.int32),
            pltpu.SemaphoreType.DMA,
            pltpu.SemaphoreType.DMA,
            pltpu.SemaphoreType.DMA,
        ],
    )
    def k(nf_hbm, src_hbm, dst_hbm, ew_hbm, zeros_hbm, out_hbm,
          acc_sh, nfb, ewb, sidx, didx, gsem, esem, ssem):
        sid = lax.axis_index("s")
        cid = lax.axis_index("c")
        wid = sid * NC + cid
        base = pl.multiple_of(wid * per_w, 8)
        tbase = pl.multiple_of(sid * rows_per_tile, 8)
        pltpu.sync_copy(zeros_hbm.at[pl.ds(tbase, rows_per_tile)],
                        acc_sh.at[pl.ds(tbase, rows_per_tile)])

        pltpu.sync_copy(src_hbm.at[wid, 0], sidx.at[0])
        pltpu.sync_copy(dst_hbm.at[wid, 0], didx.at[0])
        plsc.subcore_barrier()
        pltpu.async_copy(nf_hbm.at[sidx.at[0, 0]], nfb.at[0], gsem)
        pltpu.async_copy(ew_hbm.at[pl.ds(base, CH)], ewb.at[0], esem)

        def body(ck, carry):
            slot = lax.rem(ck, 2)
            nxt = lax.rem(ck + 1, 2)
            nb = (ck + 1) // SBN
            nbs = lax.rem(nb, 2)

            @pl.when(ck >= 1)
            def _():
                pltpu.make_async_copy(
                    nfb.at[nxt], acc_sh.at[pl.ds(0, CH)], ssem).wait()

            @pl.when(jnp.logical_and(lax.rem(ck + 1, SBN) == 0,
                                     ck + 1 < n_chunks))
            def _():
                pltpu.sync_copy(src_hbm.at[wid, nb], sidx.at[nbs])
                pltpu.sync_copy(dst_hbm.at[wid, nb], didx.at[nbs])

            @pl.when(ck + 1 < n_chunks)
            def _():
                pltpu.async_copy(
                    nf_hbm.at[sidx.at[nbs, lax.rem(ck + 1, SBN)]],
                    nfb.at[nxt], gsem)
                pltpu.async_copy(ew_hbm.at[pl.ds(base + (ck + 1) * CH, CH)],
                                 ewb.at[nxt], esem)

            pltpu.make_async_copy(nf_hbm.at[sidx.at[0, 0]], nfb.at[slot],
                                  gsem).wait()
            pltpu.make_async_copy(ew_hbm.at[pl.ds(base, CH)], ewb.at[slot],
                                  esem).wait()
            _mul_rows(nfb.at[slot], nfb.at[slot], ewb.at[slot], CH)
            pltpu.make_async_copy(
                nfb.at[slot],
                acc_sh.at[didx.at[lax.rem(ck // SBN, 2), lax.rem(ck, SBN)]],
                ssem).start(add=True)
            return carry

        lax.fori_loop(0, n_chunks, body, 0)
        pltpu.make_async_copy(nfb.at[0], acc_sh.at[pl.ds(0, CH)], ssem).wait()
        plsc.subcore_barrier()
        pltpu.sync_copy(acc_sh.at[pl.ds(tbase, rows_per_tile)],
                        out_hbm.at[cid, pl.ds(tbase, rows_per_tile)])

    return k


def _make_sc_pair_add():
    per_w = E // NW
    n_chunks = per_w // CH

    @functools.partial(
        pl.kernel,
        out_type=jax.ShapeDtypeStruct((E, D), jnp.float32),
        mesh=plsc.VectorSubcoreMesh(**_MESH),
        scratch_types=[
            pltpu.VMEM((2, CH, D), jnp.float32),
            pltpu.VMEM((2, CH, D), jnp.float32),
            pltpu.VMEM((n_chunks, CH), jnp.int32),
            pltpu.VMEM((n_chunks, CH), jnp.int32),
            pltpu.SemaphoreType.DMA,
            pltpu.SemaphoreType.DMA,
            pltpu.SemaphoreType.DMA,
        ],
    )
    def k(g_hbm, src_hbm, dst_hbm, out_hbm, sb, db, sidx, didx,
          s_sem, d_sem, osem):
        wid = lax.axis_index("s") * NC + lax.axis_index("c")
        base = pl.multiple_of(wid * per_w, 8)
        pltpu.sync_copy(src_hbm.at[wid], sidx)
        pltpu.sync_copy(dst_hbm.at[wid], didx)

        pltpu.async_copy(g_hbm.at[sidx.at[0]], sb.at[0], s_sem)
        pltpu.async_copy(g_hbm.at[didx.at[0]], db.at[0], d_sem)

        def body(ck, carry):
            slot = lax.rem(ck, 2)
            nxt = lax.rem(ck + 1, 2)

            @pl.when(jnp.logical_and(ck >= 1, ck + 1 < n_chunks))
            def _():
                pltpu.make_async_copy(
                    sb.at[nxt], out_hbm.at[pl.ds(base, CH)], osem).wait()

            @pl.when(ck + 1 < n_chunks)
            def _():
                pltpu.async_copy(g_hbm.at[sidx.at[ck + 1]], sb.at[nxt], s_sem)
                pltpu.async_copy(g_hbm.at[didx.at[ck + 1]], db.at[nxt], d_sem)

            pltpu.make_async_copy(g_hbm.at[sidx.at[ck]], sb.at[slot],
                                  s_sem).wait()
            pltpu.make_async_copy(g_hbm.at[didx.at[ck]], db.at[slot],
                                  d_sem).wait()
            _add_rows(sb.at[slot], sb.at[slot], db.at[slot], CH)
            pltpu.async_copy(sb.at[slot],
                             out_hbm.at[pl.ds(base + ck * CH, CH)], osem)
            return carry

        lax.fori_loop(0, n_chunks, body, 0)
        pltpu.make_async_copy(sb.at[0], out_hbm.at[pl.ds(base, CH)],
                              osem).wait()
        pltpu.make_async_copy(sb.at[1], out_hbm.at[pl.ds(base, CH)],
                              osem).wait()

    return k


_LOG2E = 1.4426950408889634
_COS_C = (0.0016053627764966202, -0.02539111138418885, 0.2350633717632542,
          -1.3351744534108685, 4.058698262269186, -4.934801388370931,
          0.9999999922898464)


def _rbf_from_d2(d2, vs=None):
    be = d2.shape[0]
    d2b = jnp.broadcast_to(d2, (be, NB))
    distb = jnp.sqrt(d2b + 1e-8)
    centers = lax.broadcasted_iota(jnp.int32, (1, NB), 1).astype(jnp.float32) * (
        CUTOFF / (NB - 1))
    a = distb - centers
    g = jnp.exp2(a * a * (-2.0 * _LOG2E))
    t = jnp.minimum(distb, CUTOFF) * (1.0 / CUTOFF)
    u = t * t
    c = _COS_C[0]
    for coef in _COS_C[1:]:
        c = c * u + coef
    rbf = g * (0.5 * (c + 1.0))
    if vs is not None:
        rbf = rbf * (1.0 + jnp.broadcast_to(vs, (be, NB)) / (3.0 * distb))
    return rbf


def _ew_body(d2_ref, vs_ref, w_ref, ew0_ref, ew1_ref, ew2_ref):
    rbf = _rbf_from_d2(d2_ref[...], vs_ref[...])
    ew0_ref[...] = jnp.dot(rbf, w_ref[0], preferred_element_type=jnp.float32)
    ew1_ref[...] = jnp.dot(rbf, w_ref[1], preferred_element_type=jnp.float32)
    ew2_ref[...] = jnp.dot(rbf, w_ref[2], preferred_element_type=jnp.float32)


def _tc_ew_all(d2, vs, w3, be=2000):
    grid = (E // be,)
    eblk = pl.BlockSpec((be, D), lambda b: (b, 0))
    sd = jax.ShapeDtypeStruct((E, D), jnp.float32)
    return pl.pallas_call(
        _ew_body,
        grid=grid,
        in_specs=[
            pl.BlockSpec((be, 1), lambda b: (b, 0)),
            pl.BlockSpec((be, 1), lambda b: (b, 0)),
            pl.BlockSpec((3, NB, D), lambda b: (0, 0, 0)),
        ],
        out_specs=(eblk, eblk, eblk),
        out_shape=(sd, sd, sd),
    )(d2, vs, w3)


def _update_body(nf_ref, agg_ref, ws_ref, wm_ref, out_ref):
    agg = agg_ref[0] + agg_ref[1]
    h = (jnp.dot(nf_ref[...], ws_ref[...], preferred_element_type=jnp.float32)
         + jnp.dot(agg, wm_ref[...], preferred_element_type=jnp.float32))
    out_ref[...] = _silu(h)


def _update_ext_body(nf_ref, agg_ref, ws_ref, wm_ref, wii_ref, wij_ref,
                     out_ref, g_ref, fii_ref):
    agg = agg_ref[0] + agg_ref[1]
    h = (jnp.dot(nf_ref[...], ws_ref[...], preferred_element_type=jnp.float32)
         + jnp.dot(agg, wm_ref[...], preferred_element_type=jnp.float32))
    nf = _silu(h)
    out_ref[...] = nf
    g_ref[...] = jnp.dot(nf, wij_ref[...], preferred_element_type=jnp.float32)
    t = jnp.dot(nf, wii_ref[...], preferred_element_type=jnp.float32)
    fii_ref[...] = _silu(t)


def _tc_update(nf, aggp, ws, wm, bn=2000):
    grid = (N // bn,)
    blk = pl.BlockSpec((bn, D), lambda b: (b, 0))
    ablk = pl.BlockSpec((NC, bn, D), lambda b: (0, b, 0))
    wblk = pl.BlockSpec((D, D), lambda b: (0, 0))
    return pl.pallas_call(
        _update_body, grid=grid,
        in_specs=[blk, ablk, wblk, wblk],
        out_specs=blk,
        out_shape=jax.ShapeDtypeStruct((N, D), jnp.float32),
    )(nf, aggp, ws, wm)


def _tc_update_ext(nf, aggp, ws, wm, wii, wij, bn=2000):
    grid = (N // bn,)
    blk = pl.BlockSpec((bn, D), lambda b: (b, 0))
    ablk = pl.BlockSpec((NC, bn, D), lambda b: (0, b, 0))
    wblk = pl.BlockSpec((D, D), lambda b: (0, 0))
    sd = jax.ShapeDtypeStruct((N, D), jnp.float32)
    return pl.pallas_call(
        _update_ext_body, grid=grid,
        in_specs=[blk, ablk, wblk, wblk, wblk, wblk],
        out_specs=(blk, blk, blk),
        out_shape=(sd, sd, sd),
    )(nf, aggp, ws, wm, wii, wij)


def _silu(h):
    return h / (1.0 + jnp.exp2(h * (-_LOG2E)))


def _offdiag_body0(d2_ref, h_ref, wrbf_ref, wout_ref, out_ref):
    rbf = _rbf_from_d2(d2_ref[...])
    ew = jnp.dot(rbf, wrbf_ref[...], preferred_element_type=jnp.float32)
    h = _silu(h_ref[...]) * ew
    out_ref[...] = jnp.dot(h, wout_ref[...], preferred_element_type=jnp.float32)


def _offdiag_body1(d2_ref, h_ref, wrbf_ref, wout_ref, prev_ref, out_ref):
    rbf = _rbf_from_d2(d2_ref[...])
    ew = jnp.dot(rbf, wrbf_ref[...], preferred_element_type=jnp.float32)
    h = _silu(h_ref[...]) * ew
    out_ref[...] = prev_ref[...] + jnp.dot(
        h, wout_ref[...], preferred_element_type=jnp.float32)


def _tc_offdiag(d2f, h, wrbf, wout, prev=None, be=2000):
    grid = (E // be,)
    specs = [
        pl.BlockSpec((be, 1), lambda b: (b, 0)),
        pl.BlockSpec((be, D), lambda b: (b, 0)),
        pl.BlockSpec((NB, D), lambda b: (0, 0)),
        pl.BlockSpec((D, OUT), lambda b: (0, 0)),
    ]
    args = [d2f, h, wrbf, wout]
    body = _offdiag_body0
    if prev is not None:
        specs.append(pl.BlockSpec((be, OUT), lambda b: (b, 0)))
        args.append(prev)
        body = _offdiag_body1
    return pl.pallas_call(
        body, grid=grid,
        in_specs=specs,
        out_specs=pl.BlockSpec((be, OUT), lambda b: (b, 0)),
        out_shape=jax.ShapeDtypeStruct((E, OUT), jnp.float32),
    )(*args)


def _diag_body(f0_ref, f1_ref, n0_ref, w_ref, out_ref):
    s = f0_ref[...] + f1_ref[...] + n0_ref[...]
    out_ref[...] = jnp.dot(s, w_ref[...], preferred_element_type=jnp.float32)


def _tc_diag(f0, f1, n0, w, bn=2000):
    grid = (N // bn,)
    blk = pl.BlockSpec((bn, D), lambda b: (b, 0))
    return pl.pallas_call(
        _diag_body, grid=grid,
        in_specs=[blk, blk, blk, pl.BlockSpec((D, OUT), lambda b: (0, 0))],
        out_specs=pl.BlockSpec((bn, OUT), lambda b: (b, 0)),
        out_shape=jax.ShapeDtypeStruct((N, OUT), jnp.float32),
    )(f0, f1, n0, w)


def kernel(at_no, pos, edge_index, edge_index_full, embed_table, conv_Wrbf,
           conv_Wself, conv_Wmsg, trans_Wii, trans_Wrbf, trans_Wij,
           out_Wii, out_Wij):
    f32 = jnp.float32
    src = edge_index[0].astype(jnp.int32)
    dst = edge_index[1].astype(jnp.int32)
    src_f = edge_index_full[0].astype(jnp.int32)
    dst_f = edge_index_full[1].astype(jnp.int32)

    posf = pos.astype(f32)
    zeros_nd = jnp.zeros((NACC, D), f32)
    src4d = src.reshape(NW, -1, SBN, CH)
    dst4d = dst.reshape(NW, -1, SBN, CH)
    srcf3d = src_f.reshape(NW, -1, CH)
    dstf3d = dst_f.reshape(NW, -1, CH)
    at3d = jnp.pad(at_no.astype(jnp.int32), (0, NPAD - N)).reshape(NW, -1, CH)

    d2, vs, d2f, nf0p = _make_sc_geo_embed()(
        posf[:, 0], posf[:, 1], posf[:, 2], src, dst, src_f, dst_f,
        embed_table.astype(f32), at3d)
    nf0 = nf0p[:N]
    d2 = d2.reshape(E, 1)
    vs = vs.reshape(E, 1)
    d2f = d2f.reshape(E, 1)

    ews = _tc_ew_all(d2, vs, conv_Wrbf.astype(f32))
    pair_add = _make_sc_pair_add()
    conv = _make_sc_conv()

    nf = nf0
    fii = []
    offd = None
    for i in range(3):
        aggp = conv(nf, src4d, dst4d, ews[i], zeros_nd)
        if i == 0:
            nf = _tc_update(nf, aggp, conv_Wself[i].astype(f32),
                            conv_Wmsg[i].astype(f32))
        else:
            j = i - 1
            nf, g, fii_j = _tc_update_ext(
                nf, aggp, conv_Wself[i].astype(f32), conv_Wmsg[i].astype(f32),
                trans_Wii[j].astype(f32), trans_Wij[j].astype(f32))
            fii.append(fii_j)
            h = pair_add(g, srcf3d, dstf3d)
            offd = _tc_offdiag(d2f, h, trans_Wrbf[j].astype(f32),
                               out_Wij.astype(f32), offd)

    diag = _tc_diag(fii[0], fii[1], nf0, out_Wii.astype(f32))
    return (diag, offd)


# speedup vs baseline: 6.1133x; 1.3893x over previous
import functools

import jax
import jax.numpy as jnp
from jax import lax
from jax.experimental import pallas as pl
from jax.experimental.pallas import tpu as pltpu
from jax.experimental.pallas import tpu_sc as plsc

N = 10000
E = 320000
D = 128
NB = 32
OUT = 64
CUTOFF = 5.0

NC = 2
NS = 16
NW = NC * NS
CH = 80
NACC = 10240
NPAD = NW * CH * 4

_MESH = dict(core_axis_name="c", subcore_axis_name="s")
_NOLAYOUT = pltpu.CompilerParams(needs_layout_passes=False)


def _mul_rows(dst_ref, a_ref, b_ref, n_rows):
    @plsc.parallel_loop(0, n_rows, 1, unroll=4)
    def row(r):
        for c8 in range(D // 16):
            s = pl.ds(c8 * 16, 16)
            dst_ref[r, s] = a_ref[r, s] * b_ref[r, s]


def _add_rows(dst_ref, a_ref, b_ref, n_rows):
    @plsc.parallel_loop(0, n_rows, 1, unroll=4)
    def row(r):
        for c8 in range(D // 16):
            s = pl.ds(c8 * 16, 16)
            dst_ref[r, s] = a_ref[r, s] + b_ref[r, s]


def _make_sc_geo_embed():
    per_w = E // NW
    GCH = 2000
    g_chunks = per_w // GCH
    iters = GCH // 16
    e_chunks = NPAD // (NW * CH)

    sd = jax.ShapeDtypeStruct((E,), jnp.float32)

    @functools.partial(
        pl.kernel,
        out_type=(sd, sd, sd, jax.ShapeDtypeStruct((NPAD, D), jnp.float32)),
        mesh=plsc.VectorSubcoreMesh(**_MESH),
        compiler_params=_NOLAYOUT,
        scratch_types=[
            pltpu.VMEM((N,), jnp.float32),
            pltpu.VMEM((N,), jnp.float32),
            pltpu.VMEM((N,), jnp.float32),
            pltpu.VMEM((GCH,), jnp.int32),
            pltpu.VMEM((GCH,), jnp.int32),
            pltpu.VMEM((GCH,), jnp.float32),
            pltpu.VMEM((GCH,), jnp.float32),
            pltpu.VMEM((e_chunks, CH), jnp.int32),
            pltpu.VMEM((CH, D), jnp.float32),
            pltpu.SemaphoreType.DMA,
        ],
    )
    def k(px_h, py_h, pz_h, src_h, dst_h, srcf_h, dstf_h, emb_h, at_h,
          d2_h, vs_h, d2f_h, nf0_h,
          px, py, pz, si, di, d2v, vsv, eidx, ebuf, sem):
        wid = lax.axis_index("s") * NC + lax.axis_index("c")
        base = pl.multiple_of(wid * per_w, 8)

        ebase = pl.multiple_of(wid * e_chunks * CH, 8)
        pltpu.sync_copy(at_h.at[wid], eidx)
        for ck in range(e_chunks):
            pltpu.async_copy(emb_h.at[eidx.at[ck]], ebuf, sem).wait()
            pltpu.sync_copy(ebuf, nf0_h.at[pl.ds(ebase + ck * CH, CH)])

        pltpu.sync_copy(px_h, px)
        pltpu.sync_copy(py_h, py)
        pltpu.sync_copy(pz_h, pz)

        def run(src_ref, dst_ref, out_d2, out_vs):
            def chunk(gc, carry):
                cbase = pl.multiple_of(base + gc * GCH, 8)
                pltpu.sync_copy(src_ref.at[pl.ds(cbase, GCH)], si)
                pltpu.sync_copy(dst_ref.at[pl.ds(cbase, GCH)], di)

                @plsc.parallel_loop(0, iters, 1, unroll=4)
                def body(i):
                    off = pl.multiple_of(i * 16, 8)
                    s16 = si[pl.ds(off, 16)]
                    d16 = di[pl.ds(off, 16)]
                    vx = plsc.load_gather(px, [d16]) - plsc.load_gather(px, [s16])
                    vy = plsc.load_gather(py, [d16]) - plsc.load_gather(py, [s16])
                    vz = plsc.load_gather(pz, [d16]) - plsc.load_gather(pz, [s16])
                    d2v[pl.ds(off, 16)] = vx * vx + vy * vy + vz * vz
                    if out_vs is not None:
                        vsv[pl.ds(off, 16)] = vx + vy + vz
                pltpu.sync_copy(d2v, out_d2.at[pl.ds(cbase, GCH)])
                if out_vs is not None:
                    pltpu.sync_copy(vsv, out_vs.at[pl.ds(cbase, GCH)])
                return carry
            lax.fori_loop(0, g_chunks, chunk, 0)

        run(src_h, dst_h, d2_h, vs_h)
        run(srcf_h, dstf_h, d2f_h, None)

    return k


SBN = 5


def _make_sc_conv():
    per_w = E // NW
    n_chunks = per_w // CH
    rows_per_tile = NACC // NS

    @functools.partial(
        pl.kernel,
        out_type=jax.ShapeDtypeStruct((NC, NACC, D), jnp.float32),
        mesh=plsc.VectorSubcoreMesh(**_MESH),
        scratch_types=[
            pltpu.VMEM_SHARED((NACC, D), jnp.float32),
            pltpu.VMEM((2, CH, D), jnp.float32),
            pltpu.VMEM((2, CH, D), jnp.float32),
            pltpu.VMEM((2, SBN, CH), jnp.int32),
            pltpu.VMEM((2, SBN, CH), jnp.int32),
            pltpu.SemaphoreType.DMA,
            pltpu.SemaphoreType.DMA,
            pltpu.SemaphoreType.DMA,
        ],
    )
    def k(nf_hbm, src_hbm, dst_hbm, ew_hbm, zeros_hbm, out_hbm,
          acc_sh, nfb, ewb, sidx, didx, gsem, esem, ssem):
        sid = lax.axis_index("s")
        cid = lax.axis_index("c")
        wid = sid * NC + cid
        base = pl.multiple_of(wid * per_w, 8)
        tbase = pl.multiple_of(sid * rows_per_tile, 8)
        pltpu.sync_copy(zeros_hbm.at[pl.ds(tbase, rows_per_tile)],
                        acc_sh.at[pl.ds(tbase, rows_per_tile)])

        pltpu.sync_copy(src_hbm.at[wid, 0], sidx.at[0])
        pltpu.sync_copy(dst_hbm.at[wid, 0], didx.at[0])
        plsc.subcore_barrier()
        pltpu.async_copy(nf_hbm.at[sidx.at[0, 0]], nfb.at[0], gsem)
        pltpu.async_copy(ew_hbm.at[pl.ds(base, CH)], ewb.at[0], esem)

        def body(ck, carry):
            slot = lax.rem(ck, 2)
            nxt = lax.rem(ck + 1, 2)
            nb = (ck + 1) // SBN
            nbs = lax.rem(nb, 2)

            @pl.when(ck >= 1)
            def _():
                pltpu.make_async_copy(
                    nfb.at[nxt], acc_sh.at[pl.ds(0, CH)], ssem).wait()

            @pl.when(jnp.logical_and(lax.rem(ck + 1, SBN) == 0,
                                     ck + 1 < n_chunks))
            def _():
                pltpu.sync_copy(src_hbm.at[wid, nb], sidx.at[nbs])
                pltpu.sync_copy(dst_hbm.at[wid, nb], didx.at[nbs])

            @pl.when(ck + 1 < n_chunks)
            def _():
                pltpu.async_copy(
                    nf_hbm.at[sidx.at[nbs, lax.rem(ck + 1, SBN)]],
                    nfb.at[nxt], gsem)
                pltpu.async_copy(ew_hbm.at[pl.ds(base + (ck + 1) * CH, CH)],
                                 ewb.at[nxt], esem)

            pltpu.make_async_copy(nf_hbm.at[sidx.at[0, 0]], nfb.at[slot],
                                  gsem).wait()
            pltpu.make_async_copy(ew_hbm.at[pl.ds(base, CH)], ewb.at[slot],
                                  esem).wait()
            _mul_rows(nfb.at[slot], nfb.at[slot], ewb.at[slot], CH)
            pltpu.make_async_copy(
                nfb.at[slot],
                acc_sh.at[didx.at[lax.rem(ck // SBN, 2), lax.rem(ck, SBN)]],
                ssem).start(add=True)
            return carry

        lax.fori_loop(0, n_chunks, body, 0)
        pltpu.make_async_copy(nfb.at[0], acc_sh.at[pl.ds(0, CH)], ssem).wait()
        plsc.subcore_barrier()
        pltpu.sync_copy(acc_sh.at[pl.ds(tbase, rows_per_tile)],
                        out_hbm.at[cid, pl.ds(tbase, rows_per_tile)])

    return k


def _make_sc_pair_add():
    per_w = E // NW
    n_chunks = per_w // CH

    @functools.partial(
        pl.kernel,
        out_type=jax.ShapeDtypeStruct((E, D), jnp.float32),
        mesh=plsc.VectorSubcoreMesh(**_MESH),
        scratch_types=[
            pltpu.VMEM((2, CH, D), jnp.float32),
            pltpu.VMEM((2, CH, D), jnp.float32),
            pltpu.VMEM((n_chunks, CH), jnp.int32),
            pltpu.VMEM((n_chunks, CH), jnp.int32),
            pltpu.SemaphoreType.DMA,
            pltpu.SemaphoreType.DMA,
            pltpu.SemaphoreType.DMA,
        ],
    )
    def k(g_hbm, src_hbm, dst_hbm, out_hbm, sb, db, sidx, didx,
          s_sem, d_sem, osem):
        wid = lax.axis_index("s") * NC + lax.axis_index("c")
        base = pl.multiple_of(wid * per_w, 8)
        pltpu.sync_copy(src_hbm.at[wid], sidx)
        pltpu.sync_copy(dst_hbm.at[wid], didx)

        pltpu.async_copy(g_hbm.at[sidx.at[0]], sb.at[0], s_sem)
        pltpu.async_copy(g_hbm.at[didx.at[0]], db.at[0], d_sem)

        def body(ck, carry):
            slot = lax.rem(ck, 2)
            nxt = lax.rem(ck + 1, 2)

            @pl.when(jnp.logical_and(ck >= 1, ck + 1 < n_chunks))
            def _():
                pltpu.make_async_copy(
                    sb.at[nxt], out_hbm.at[pl.ds(base, CH)], osem).wait()

            @pl.when(ck + 1 < n_chunks)
            def _():
                pltpu.async_copy(g_hbm.at[sidx.at[ck + 1]], sb.at[nxt], s_sem)
                pltpu.async_copy(g_hbm.at[didx.at[ck + 1]], db.at[nxt], d_sem)

            pltpu.make_async_copy(g_hbm.at[sidx.at[ck]], sb.at[slot],
                                  s_sem).wait()
            pltpu.make_async_copy(g_hbm.at[didx.at[ck]], db.at[slot],
                                  d_sem).wait()
            _add_rows(sb.at[slot], sb.at[slot], db.at[slot], CH)
            pltpu.async_copy(sb.at[slot],
                             out_hbm.at[pl.ds(base + ck * CH, CH)], osem)
            return carry

        lax.fori_loop(0, n_chunks, body, 0)
        pltpu.make_async_copy(sb.at[0], out_hbm.at[pl.ds(base, CH)],
                              osem).wait()
        pltpu.make_async_copy(sb.at[1], out_hbm.at[pl.ds(base, CH)],
                              osem).wait()

    return k


_LOG2E = 1.4426950408889634
_COS_C = (0.0016053627764966202, -0.02539111138418885, 0.2350633717632542,
          -1.3351744534108685, 4.058698262269186, -4.934801388370931,
          0.9999999922898464)


def _rbf_t(d2row, vsrow=None):
    be = d2row.shape[1]
    d2b = jnp.broadcast_to(d2row, (NB, be))
    distb = jnp.sqrt(d2b + 1e-8)
    centers = lax.broadcasted_iota(jnp.int32, (NB, 1), 0).astype(jnp.float32) * (
        CUTOFF / (NB - 1))
    a = distb - centers
    g = jnp.exp2(a * a * (-2.0 * _LOG2E))
    t = jnp.minimum(distb, CUTOFF) * (1.0 / CUTOFF)
    u = t * t
    c = _COS_C[0]
    for coef in _COS_C[1:]:
        c = c * u + coef
    rbf = g * (0.5 * (c + 1.0))
    if vsrow is not None:
        rbf = rbf * (1.0 + jnp.broadcast_to(vsrow, (NB, be)) / (3.0 * distb))
    return rbf


_DN_T = (((0,), (0,)), ((), ()))


def _ew_body(d2_ref, vs_ref, w_ref, ew0_ref, ew1_ref, ew2_ref):
    rbf = _rbf_t(d2_ref[0], vs_ref[0])
    ew0_ref[...] = lax.dot_general(rbf, w_ref[0], _DN_T,
                                   preferred_element_type=jnp.float32)
    ew1_ref[...] = lax.dot_general(rbf, w_ref[1], _DN_T,
                                   preferred_element_type=jnp.float32)
    ew2_ref[...] = lax.dot_general(rbf, w_ref[2], _DN_T,
                                   preferred_element_type=jnp.float32)


def _tc_ew_all(d2, vs, w3, be=2560):
    grid = (E // be,)
    eblk = pl.BlockSpec((be, D), lambda b: (b, 0))
    sd = jax.ShapeDtypeStruct((E, D), jnp.float32)
    return pl.pallas_call(
        _ew_body,
        grid=grid,
        in_specs=[
            pl.BlockSpec((1, 1, be), lambda b: (b, 0, 0)),
            pl.BlockSpec((1, 1, be), lambda b: (b, 0, 0)),
            pl.BlockSpec((3, NB, D), lambda b: (0, 0, 0)),
        ],
        out_specs=(eblk, eblk, eblk),
        out_shape=(sd, sd, sd),
    )(d2.reshape(E // be, 1, be), vs.reshape(E // be, 1, be), w3)


def _update_body(nf_ref, agg_ref, ws_ref, wm_ref, out_ref):
    agg = agg_ref[0] + agg_ref[1]
    h = (jnp.dot(nf_ref[...], ws_ref[...], preferred_element_type=jnp.float32)
         + jnp.dot(agg, wm_ref[...], preferred_element_type=jnp.float32))
    out_ref[...] = _silu(h)


def _update_ext_body(nf_ref, agg_ref, ws_ref, wm_ref, wii_ref, wij_ref,
                     out_ref, g_ref, fii_ref):
    agg = agg_ref[0] + agg_ref[1]
    h = (jnp.dot(nf_ref[...], ws_ref[...], preferred_element_type=jnp.float32)
         + jnp.dot(agg, wm_ref[...], preferred_element_type=jnp.float32))
    nf = _silu(h)
    out_ref[...] = nf
    g_ref[...] = jnp.dot(nf, wij_ref[...], preferred_element_type=jnp.float32)
    t = jnp.dot(nf, wii_ref[...], preferred_element_type=jnp.float32)
    fii_ref[...] = _silu(t)


def _tc_update(nf, aggp, ws, wm, bn=2000):
    grid = (N // bn,)
    blk = pl.BlockSpec((bn, D), lambda b: (b, 0))
    ablk = pl.BlockSpec((NC, bn, D), lambda b: (0, b, 0))
    wblk = pl.BlockSpec((D, D), lambda b: (0, 0))
    return pl.pallas_call(
        _update_body, grid=grid,
        in_specs=[blk, ablk, wblk, wblk],
        out_specs=blk,
        out_shape=jax.ShapeDtypeStruct((N, D), jnp.float32),
    )(nf, aggp, ws, wm)


def _tc_update_ext(nf, aggp, ws, wm, wii, wij, bn=2000):
    grid = (N // bn,)
    blk = pl.BlockSpec((bn, D), lambda b: (b, 0))
    ablk = pl.BlockSpec((NC, bn, D), lambda b: (0, b, 0))
    wblk = pl.BlockSpec((D, D), lambda b: (0, 0))
    sd = jax.ShapeDtypeStruct((N, D), jnp.float32)
    return pl.pallas_call(
        _update_ext_body, grid=grid,
        in_specs=[blk, ablk, wblk, wblk, wblk, wblk],
        out_specs=(blk, blk, blk),
        out_shape=(sd, sd, sd),
    )(nf, aggp, ws, wm, wii, wij)


def _silu(h):
    return h / (1.0 + jnp.exp2(h * (-_LOG2E)))


def _offdiag_body0(d2_ref, h_ref, wrbf_ref, wout_ref, out_ref):
    rbf = _rbf_t(d2_ref[0])
    ew = lax.dot_general(rbf, wrbf_ref[...], _DN_T,
                         preferred_element_type=jnp.float32)
    h = _silu(h_ref[...]) * ew
    out_ref[...] = jnp.dot(h, wout_ref[...], preferred_element_type=jnp.float32)


def _offdiag_body1(d2_ref, h_ref, wrbf_ref, wout_ref, prev_ref, out_ref):
    rbf = _rbf_t(d2_ref[0])
    ew = lax.dot_general(rbf, wrbf_ref[...], _DN_T,
                         preferred_element_type=jnp.float32)
    h = _silu(h_ref[...]) * ew
    out_ref[...] = prev_ref[...] + jnp.dot(
        h, wout_ref[...], preferred_element_type=jnp.float32)


def _tc_offdiag(d2f, h, wrbf, wout, prev=None, be=2560):
    grid = (E // be,)
    specs = [
        pl.BlockSpec((1, 1, be), lambda b: (b, 0, 0)),
        pl.BlockSpec((be, D), lambda b: (b, 0)),
        pl.BlockSpec((NB, D), lambda b: (0, 0)),
        pl.BlockSpec((D, OUT), lambda b: (0, 0)),
    ]
    args = [d2f.reshape(E // be, 1, be), h, wrbf, wout]
    body = _offdiag_body0
    if prev is not None:
        specs.append(pl.BlockSpec((be, OUT), lambda b: (b, 0)))
        args.append(prev)
        body = _offdiag_body1
    return pl.pallas_call(
        body, grid=grid,
        in_specs=specs,
        out_specs=pl.BlockSpec((be, OUT), lambda b: (b, 0)),
        out_shape=jax.ShapeDtypeStruct((E, OUT), jnp.float32),
    )(*args)


def _diag_body(f0_ref, f1_ref, n0_ref, w_ref, out_ref):
    s = f0_ref[...] + f1_ref[...] + n0_ref[...]
    out_ref[...] = jnp.dot(s, w_ref[...], preferred_element_type=jnp.float32)


def _tc_diag(f0, f1, n0, w, bn=2000):
    grid = (N // bn,)
    blk = pl.BlockSpec((bn, D), lambda b: (b, 0))
    return pl.pallas_call(
        _diag_body, grid=grid,
        in_specs=[blk, blk, blk, pl.BlockSpec((D, OUT), lambda b: (0, 0))],
        out_specs=pl.BlockSpec((bn, OUT), lambda b: (b, 0)),
        out_shape=jax.ShapeDtypeStruct((N, OUT), jnp.float32),
    )(f0, f1, n0, w)


def kernel(at_no, pos, edge_index, edge_index_full, embed_table, conv_Wrbf,
           conv_Wself, conv_Wmsg, trans_Wii, trans_Wrbf, trans_Wij,
           out_Wii, out_Wij):
    f32 = jnp.float32
    src = edge_index[0].astype(jnp.int32)
    dst = edge_index[1].astype(jnp.int32)
    src_f = edge_index_full[0].astype(jnp.int32)
    dst_f = edge_index_full[1].astype(jnp.int32)

    posf = pos.astype(f32)
    zeros_nd = jnp.zeros((NACC, D), f32)
    src4d = src.reshape(NW, -1, SBN, CH)
    dst4d = dst.reshape(NW, -1, SBN, CH)
    srcf3d = src_f.reshape(NW, -1, CH)
    dstf3d = dst_f.reshape(NW, -1, CH)
    at3d = jnp.pad(at_no.astype(jnp.int32), (0, NPAD - N)).reshape(NW, -1, CH)

    d2, vs, d2f, nf0p = _make_sc_geo_embed()(
        posf[:, 0], posf[:, 1], posf[:, 2], src, dst, src_f, dst_f,
        embed_table.astype(f32), at3d)
    nf0 = nf0p[:N]

    ews = _tc_ew_all(d2, vs, conv_Wrbf.astype(f32))
    pair_add = _make_sc_pair_add()
    conv = _make_sc_conv()

    nf = nf0
    fii = []
    offd = None
    for i in range(3):
        aggp = conv(nf, src4d, dst4d, ews[i], zeros_nd)
        if i == 0:
            nf = _tc_update(nf, aggp, conv_Wself[i].astype(f32),
                            conv_Wmsg[i].astype(f32))
        else:
            j = i - 1
            nf, g, fii_j = _tc_update_ext(
                nf, aggp, conv_Wself[i].astype(f32), conv_Wmsg[i].astype(f32),
                trans_Wii[j].astype(f32), trans_Wij[j].astype(f32))
            fii.append(fii_j)
            h = pair_add(g, srcf3d, dstf3d)
            offd = _tc_offdiag(d2f, h, trans_Wrbf[j].astype(f32),
                               out_Wij.astype(f32), offd)

    diag = _tc_diag(fii[0], fii[1], nf0, out_Wii.astype(f32))
    return (diag, offd)
